# Initial kernel scaffold; baseline (speedup 1.0000x reference)
import jax
import jax.numpy as jnp
from jax.experimental import pallas as pl


def kernel(x, edge_index, batch, emb_table, W0, b0, g0, be0, W1, b1, g1, be1):
    raise NotImplementedError("write your pallas kernel here")


# speedup vs baseline: 8.8115x; 8.8115x over previous
import functools

import jax
import jax.numpy as jnp
from jax import lax
from jax.experimental import pallas as pl
from jax.experimental.pallas import tpu as pltpu
from jax.experimental.pallas import tpu_sc as plsc

N = 10000
D = 128
E = 320000
G = 16
CARD = 128
EPS = 1e-5

NC, NS = 2, 16
NW = NC * NS
CH = 128
NCH = 80
EPW = CH * NCH
EPAD = EPW * NW
RPT = 632
NPAD = RPT * NS
HNCH = NCH // 2

R = 1000
NBLK = N // R

@functools.cache
def _mesh():
    return plsc.VectorSubcoreMesh(core_axis_name="c", subcore_axis_name="s",
                                  num_cores=NC, num_subcores=NS)


def _sc_deg_body(dst_hbm, ones_hbm, zero_hbm, out_hbm, dst_v, ones_v, deg_sh):
    c = lax.axis_index("c")
    s = lax.axis_index("s")
    wid = c * NS + s

    @pl.when(s == 0)
    def _():
        pltpu.sync_copy(zero_hbm, deg_sh)

    pltpu.sync_copy(ones_hbm, ones_v)
    pltpu.sync_copy(dst_hbm.at[wid], dst_v)
    plsc.subcore_barrier()

    def _chunk(j, _):
        pltpu.sync_copy(ones_v, deg_sh.at[dst_v.at[j]], add=True)
        return 0

    lax.fori_loop(0, NCH, _chunk, 0)
    plsc.subcore_barrier()

    @pl.when(s == 0)
    def _():
        pltpu.sync_copy(deg_sh, out_hbm.at[c])


@functools.cache
def _sc_deg_kernel():
    return pl.kernel(
        _sc_deg_body,
        out_type=jax.ShapeDtypeStruct((NC, NPAD, D), jnp.float32),
        mesh=_mesh(),
        scratch_types=[
            pltpu.VMEM((NCH, CH), jnp.int32),
            pltpu.VMEM((CH, D), jnp.float32),
            pltpu.VMEM_SHARED((NPAD, D), jnp.float32),
        ],
    )


def _sc_deg(dst3):
    ones_d = jnp.ones((CH, D), jnp.float32)
    zero_d = jnp.zeros((NPAD, D), jnp.float32)
    return _sc_deg_kernel()(dst3, ones_d, zero_d)


def _sc_scat_body(y_hbm, src_hbm, dst_hbm, zero_hbm, out_hbm,
                  rows_a, src_v, dst_v, z_sh, sem_a):
    c = lax.axis_index("c")
    s = lax.axis_index("s")
    wid = c * NS + s

    @pl.when(s == 0)
    def _():
        pltpu.sync_copy(zero_hbm, z_sh)

    pltpu.sync_copy(src_hbm.at[wid], src_v)
    pltpu.sync_copy(dst_hbm.at[wid], dst_v)
    plsc.subcore_barrier()

    def _step(j, _):
        pltpu.async_copy(y_hbm.at[src_v.at[j]], rows_a, sem_a).wait()
        pltpu.sync_copy(rows_a, z_sh.at[dst_v.at[j]], add=True)
        return 0

    lax.fori_loop(0, NCH, _step, 0)
    plsc.subcore_barrier()

    @pl.when(s == 0)
    def _():
        pltpu.sync_copy(z_sh, out_hbm.at[c])


@functools.cache
def _sc_scatter_kernel():
    return pl.kernel(
        _sc_scat_body,
        out_type=jax.ShapeDtypeStruct((NC, NPAD, D), jnp.float32),
        mesh=_mesh(),
        scratch_types=[
            pltpu.VMEM((CH, D), jnp.float32),
            pltpu.VMEM((NCH, CH), jnp.int32),
            pltpu.VMEM((NCH, CH), jnp.int32),
            pltpu.VMEM_SHARED((NPAD, D), jnp.float32),
            pltpu.SemaphoreType.DMA,
        ],
    )


def _sc_scatter(y, src3, dst3):
    zero_d = jnp.zeros((NPAD, D), jnp.float32)
    return _sc_scatter_kernel()(y, src3, dst3, zero_d)


def _tc_a_body(x_ref, emb_ref, w0_ref, degp_ref, y0_ref, dinv_ref):
    xb = x_ref[...]
    oh = (xb == lax.broadcasted_iota(jnp.int32, (R, CARD), 1))
    h0 = jnp.dot(oh.astype(jnp.float32), emb_ref[...],
                 preferred_element_type=jnp.float32)
    xw = jnp.dot(h0, w0_ref[...], preferred_element_type=jnp.float32)
    deg = degp_ref[0, :, 0:1] + degp_ref[1, :, 0:1] + 1.0
    dinv = lax.rsqrt(deg)
    dinv_ref[...] = dinv
    y0_ref[...] = xw * dinv


def _tc_a(x, emb, w0, degp):
    return pl.pallas_call(
        _tc_a_body,
        grid=(NBLK,),
        in_specs=[
            pl.BlockSpec((R, 1), lambda i: (i, 0)),
            pl.BlockSpec((CARD, D), lambda i: (0, 0)),
            pl.BlockSpec((D, D), lambda i: (0, 0)),
            pl.BlockSpec((2, R, D), lambda i: (0, i, 0)),
        ],
        out_specs=[
            pl.BlockSpec((R, D), lambda i: (i, 0)),
            pl.BlockSpec((R, 1), lambda i: (i, 0)),
        ],
        out_shape=[
            jax.ShapeDtypeStruct((N, D), jnp.float32),
            jax.ShapeDtypeStruct((N, 1), jnp.float32),
        ],
    )(x, emb, w0, degp)


def _tc_s_body(zp_ref, y_ref, dinv_ref, b_ref, agg_ref, st_ref):
    agg = (zp_ref[0] + zp_ref[1] + y_ref[...]) * dinv_ref[...] + b_ref[...]
    agg_ref[...] = agg

    @pl.when(pl.program_id(0) == 0)
    def _():
        st_ref[...] = jnp.zeros_like(st_ref)

    st_ref[0:1, :] += jnp.sum(agg, axis=0, keepdims=True)
    st_ref[1:2, :] += jnp.sum(agg * agg, axis=0, keepdims=True)


def _tc_stats(zp, y, dinv, b):
    return pl.pallas_call(
        _tc_s_body,
        grid=(NBLK,),
        in_specs=[
            pl.BlockSpec((2, R, D), lambda i: (0, i, 0)),
            pl.BlockSpec((R, D), lambda i: (i, 0)),
            pl.BlockSpec((R, 1), lambda i: (i, 0)),
            pl.BlockSpec((1, D), lambda i: (0, 0)),
        ],
        out_specs=[
            pl.BlockSpec((R, D), lambda i: (i, 0)),
            pl.BlockSpec((8, D), lambda i: (0, 0)),
        ],
        out_shape=[
            jax.ShapeDtypeStruct((N, D), jnp.float32),
            jax.ShapeDtypeStruct((8, D), jnp.float32),
        ],
    )(zp, y, dinv, b)


def _tc_m_body(agg_ref, st_ref, g_ref, be_ref, dinv_ref, w_ref, y1_ref):
    m = st_ref[0:1, :] * (1.0 / N)
    v = st_ref[1:2, :] * (1.0 / N) - m * m
    rstd = lax.rsqrt(v + EPS)
    h = jnp.maximum((agg_ref[...] - m) * rstd * g_ref[...] + be_ref[...], 0.0)
    y1_ref[...] = jnp.dot(h, w_ref[...],
                          preferred_element_type=jnp.float32) * dinv_ref[...]


def _tc_mid(agg, st, g, be, dinv, w1):
    return pl.pallas_call(
        _tc_m_body,
        grid=(NBLK,),
        in_specs=[
            pl.BlockSpec((R, D), lambda i: (i, 0)),
            pl.BlockSpec((8, D), lambda i: (0, 0)),
            pl.BlockSpec((1, D), lambda i: (0, 0)),
            pl.BlockSpec((1, D), lambda i: (0, 0)),
            pl.BlockSpec((R, 1), lambda i: (i, 0)),
            pl.BlockSpec((D, D), lambda i: (0, 0)),
        ],
        out_specs=pl.BlockSpec((R, D), lambda i: (i, 0)),
        out_shape=jax.ShapeDtypeStruct((N, D), jnp.float32),
    )(agg, st, g, be, dinv, w1)


def _tc_f_body(agg_ref, st_ref, g_ref, be_ref, batch_ref,
               nn_ref, gr_ref, cnt_scr):
    i = pl.program_id(0)
    m = st_ref[0:1, :] * (1.0 / N)
    v = st_ref[1:2, :] * (1.0 / N) - m * m
    h = (agg_ref[...] - m) * lax.rsqrt(v + EPS) * g_ref[...] + be_ref[...]
    nrm = jnp.sqrt(jnp.sum(h * h, axis=1, keepdims=True))
    nn_ref[...] = h / jnp.maximum(nrm, 1e-12)
    oh = (batch_ref[...] == lax.broadcasted_iota(jnp.int32, (R, G), 1))
    ohf = oh.astype(jnp.float32)
    part = lax.dot_general(ohf, h, (((0,), (0,)), ((), ())),
                           preferred_element_type=jnp.float32)
    pc = lax.dot_general(ohf, jnp.ones((R, D), jnp.float32),
                         (((0,), (0,)), ((), ())),
                         preferred_element_type=jnp.float32)

    @pl.when(i == 0)
    def _():
        gr_ref[...] = jnp.zeros_like(gr_ref)
        cnt_scr[...] = jnp.zeros_like(cnt_scr)

    gr_ref[...] += part
    cnt_scr[...] += pc

    @pl.when(i == NBLK - 1)
    def _():
        gr_ref[...] = gr_ref[...] / jnp.maximum(cnt_scr[...], 1.0)


def _tc_final(agg, st, g, be, batch2):
    return pl.pallas_call(
        _tc_f_body,
        grid=(NBLK,),
        in_specs=[
            pl.BlockSpec((R, D), lambda i: (i, 0)),
            pl.BlockSpec((8, D), lambda i: (0, 0)),
            pl.BlockSpec((1, D), lambda i: (0, 0)),
            pl.BlockSpec((1, D), lambda i: (0, 0)),
            pl.BlockSpec((R, 1), lambda i: (i, 0)),
        ],
        out_specs=[
            pl.BlockSpec((R, D), lambda i: (i, 0)),
            pl.BlockSpec((G, D), lambda i: (0, 0)),
        ],
        out_shape=[
            jax.ShapeDtypeStruct((N, D), jnp.float32),
            jax.ShapeDtypeStruct((G, D), jnp.float32),
        ],
        scratch_shapes=[pltpu.VMEM((G, D), jnp.float32)],
    )(agg, st, g, be, batch2)


def kernel(x, edge_index, batch, emb_table, W0, b0, g0, be0, W1, b1, g1, be1):
    src = edge_index[0]
    dst = edge_index[1]
    pad = EPAD - E
    src_p = jnp.concatenate([src, jnp.zeros((pad,), src.dtype)])
    trash = N + (jnp.arange(pad, dtype=dst.dtype) % (NPAD - N))
    dst_p = jnp.concatenate([dst, trash])
    src3 = src_p.reshape(NW, NCH, CH)
    dst3 = dst_p.reshape(NW, NCH, CH)

    degp = _sc_deg(dst3)
    y0, dinv = _tc_a(x, emb_table, W0, degp)
    z0 = _sc_scatter(y0, src3, dst3)
    agg0, st0 = _tc_stats(z0, y0, dinv, b0.reshape(1, D))
    y1 = _tc_mid(agg0, st0, g0.reshape(1, D), be0.reshape(1, D), dinv, W1)
    z1 = _sc_scatter(y1, src3, dst3)
    agg1, st1 = _tc_stats(z1, y1, dinv, b1.reshape(1, D))
    node_norm, graph_rep = _tc_final(agg1, st1, g1.reshape(1, D),
                                     be1.reshape(1, D), batch.reshape(N, 1))
    return graph_rep, node_norm


# speedup vs baseline: 9.5775x; 1.0869x over previous
import functools

import jax
import jax.numpy as jnp
from jax import lax
from jax.experimental import pallas as pl
from jax.experimental.pallas import tpu as pltpu
from jax.experimental.pallas import tpu_sc as plsc

N = 10000
D = 128
E = 320000
G = 16
CARD = 128
EPS = 1e-5

NC, NS = 2, 16
NW = NC * NS
CH = 128
NCH = 80
EPW = CH * NCH
EPAD = EPW * NW
RPT = 632
NPAD = RPT * NS
HNCH = NCH // 2

R = 1000
NBLK = N // R

@functools.cache
def _mesh():
    return plsc.VectorSubcoreMesh(core_axis_name="c", subcore_axis_name="s",
                                  num_cores=NC, num_subcores=NS)


def _sc_deg_body(dst_hbm, ones_hbm, zero_hbm, out_hbm, dst_v, ones_v, deg_sh):
    c = lax.axis_index("c")
    s = lax.axis_index("s")
    wid = c * NS + s

    @pl.when(s == 0)
    def _():
        pltpu.sync_copy(zero_hbm, deg_sh)

    pltpu.sync_copy(ones_hbm, ones_v)
    pltpu.sync_copy(dst_hbm.at[wid], dst_v)
    plsc.subcore_barrier()

    def _chunk(j, _):
        pltpu.sync_copy(ones_v, deg_sh.at[dst_v.at[j]], add=True)
        return 0

    lax.fori_loop(0, NCH, _chunk, 0)
    plsc.subcore_barrier()

    @pl.when(s == 0)
    def _():
        pltpu.sync_copy(deg_sh, out_hbm.at[c])


@functools.cache
def _sc_deg_kernel():
    return pl.kernel(
        _sc_deg_body,
        out_type=jax.ShapeDtypeStruct((NC, NPAD, D), jnp.float32),
        mesh=_mesh(),
        scratch_types=[
            pltpu.VMEM((NCH, CH), jnp.int32),
            pltpu.VMEM((CH, D), jnp.float32),
            pltpu.VMEM_SHARED((NPAD, D), jnp.float32),
        ],
    )


def _sc_deg(dst3):
    ones_d = jnp.ones((CH, D), jnp.float32)
    zero_d = jnp.zeros((NPAD, D), jnp.float32)
    return _sc_deg_kernel()(dst3, ones_d, zero_d)


def _sc_scat_body(y_hbm, src_hbm, dst_hbm, zero_hbm, out_hbm,
                  rows_a, rows_b, src_v, dst_v, z_sh, sem_a, sem_b):
    c = lax.axis_index("c")
    s = lax.axis_index("s")
    wid = c * NS + s

    @pl.when(s == 0)
    def _():
        pltpu.sync_copy(zero_hbm, z_sh)

    plsc.subcore_barrier()

    for h in range(2):
        pltpu.sync_copy(src_hbm.at[wid, pl.ds(h * HNCH, HNCH)], src_v)
        pltpu.sync_copy(dst_hbm.at[wid, pl.ds(h * HNCH, HNCH)], dst_v)
        pltpu.async_copy(y_hbm.at[src_v.at[0]], rows_a, sem_a)

        def _step(j, _):
            pltpu.make_async_copy(y_hbm.at[src_v.at[2 * j]], rows_a,
                                  sem_a).wait()
            pltpu.async_copy(y_hbm.at[src_v.at[2 * j + 1]], rows_b, sem_b)
            pltpu.sync_copy(rows_a, z_sh.at[dst_v.at[2 * j]], add=True)
            pltpu.make_async_copy(y_hbm.at[src_v.at[2 * j + 1]], rows_b,
                                  sem_b).wait()

            @pl.when(j < HNCH // 2 - 1)
            def _():
                pltpu.async_copy(y_hbm.at[src_v.at[2 * j + 2]], rows_a, sem_a)

            pltpu.sync_copy(rows_b, z_sh.at[dst_v.at[2 * j + 1]], add=True)
            return 0

        lax.fori_loop(0, HNCH // 2, _step, 0)
    plsc.subcore_barrier()

    @pl.when(s == 0)
    def _():
        pltpu.sync_copy(z_sh, out_hbm.at[c])


@functools.cache
def _sc_scatter_kernel():
    return pl.kernel(
        _sc_scat_body,
        out_type=jax.ShapeDtypeStruct((NC, NPAD, D), jnp.float32),
        mesh=_mesh(),
        scratch_types=[
            pltpu.VMEM((CH, D), jnp.float32),
            pltpu.VMEM((CH, D), jnp.float32),
            pltpu.VMEM((HNCH, CH), jnp.int32),
            pltpu.VMEM((HNCH, CH), jnp.int32),
            pltpu.VMEM_SHARED((NPAD, D), jnp.float32),
            pltpu.SemaphoreType.DMA,
            pltpu.SemaphoreType.DMA,
        ],
    )


def _sc_scatter(y, src3, dst3):
    zero_d = jnp.zeros((NPAD, D), jnp.float32)
    return _sc_scatter_kernel()(y, src3, dst3, zero_d)


def _tc_a_body(x_ref, emb_ref, w0_ref, degp_ref, y0_ref, dinv_ref):
    xb = x_ref[...]
    oh = (xb == lax.broadcasted_iota(jnp.int32, (R, CARD), 1))
    h0 = jnp.dot(oh.astype(jnp.float32), emb_ref[...],
                 preferred_element_type=jnp.float32)
    xw = jnp.dot(h0, w0_ref[...], preferred_element_type=jnp.float32)
    deg = degp_ref[0, :, 0:1] + degp_ref[1, :, 0:1] + 1.0
    dinv = lax.rsqrt(deg)
    dinv_ref[...] = dinv
    y0_ref[...] = xw * dinv


def _tc_a(x, emb, w0, degp):
    return pl.pallas_call(
        _tc_a_body,
        grid=(NBLK,),
        in_specs=[
            pl.BlockSpec((R, 1), lambda i: (i, 0)),
            pl.BlockSpec((CARD, D), lambda i: (0, 0)),
            pl.BlockSpec((D, D), lambda i: (0, 0)),
            pl.BlockSpec((2, R, D), lambda i: (0, i, 0)),
        ],
        out_specs=[
            pl.BlockSpec((R, D), lambda i: (i, 0)),
            pl.BlockSpec((R, 1), lambda i: (i, 0)),
        ],
        out_shape=[
            jax.ShapeDtypeStruct((N, D), jnp.float32),
            jax.ShapeDtypeStruct((N, 1), jnp.float32),
        ],
    )(x, emb, w0, degp)


def _tc_s_body(zp_ref, y_ref, dinv_ref, b_ref, agg_ref, st_ref):
    agg = (zp_ref[0] + zp_ref[1] + y_ref[...]) * dinv_ref[...] + b_ref[...]
    agg_ref[...] = agg

    @pl.when(pl.program_id(0) == 0)
    def _():
        st_ref[...] = jnp.zeros_like(st_ref)

    st_ref[0:1, :] += jnp.sum(agg, axis=0, keepdims=True)
    st_ref[1:2, :] += jnp.sum(agg * agg, axis=0, keepdims=True)


def _tc_stats(zp, y, dinv, b):
    return pl.pallas_call(
        _tc_s_body,
        grid=(NBLK,),
        in_specs=[
            pl.BlockSpec((2, R, D), lambda i: (0, i, 0)),
            pl.BlockSpec((R, D), lambda i: (i, 0)),
            pl.BlockSpec((R, 1), lambda i: (i, 0)),
            pl.BlockSpec((1, D), lambda i: (0, 0)),
        ],
        out_specs=[
            pl.BlockSpec((R, D), lambda i: (i, 0)),
            pl.BlockSpec((8, D), lambda i: (0, 0)),
        ],
        out_shape=[
            jax.ShapeDtypeStruct((N, D), jnp.float32),
            jax.ShapeDtypeStruct((8, D), jnp.float32),
        ],
    )(zp, y, dinv, b)


def _tc_m_body(agg_ref, st_ref, g_ref, be_ref, dinv_ref, w_ref, y1_ref):
    m = st_ref[0:1, :] * (1.0 / N)
    v = st_ref[1:2, :] * (1.0 / N) - m * m
    rstd = lax.rsqrt(v + EPS)
    h = jnp.maximum((agg_ref[...] - m) * rstd * g_ref[...] + be_ref[...], 0.0)
    y1_ref[...] = jnp.dot(h, w_ref[...],
                          preferred_element_type=jnp.float32) * dinv_ref[...]


def _tc_mid(agg, st, g, be, dinv, w1):
    return pl.pallas_call(
        _tc_m_body,
        grid=(NBLK,),
        in_specs=[
            pl.BlockSpec((R, D), lambda i: (i, 0)),
            pl.BlockSpec((8, D), lambda i: (0, 0)),
            pl.BlockSpec((1, D), lambda i: (0, 0)),
            pl.BlockSpec((1, D), lambda i: (0, 0)),
            pl.BlockSpec((R, 1), lambda i: (i, 0)),
            pl.BlockSpec((D, D), lambda i: (0, 0)),
        ],
        out_specs=pl.BlockSpec((R, D), lambda i: (i, 0)),
        out_shape=jax.ShapeDtypeStruct((N, D), jnp.float32),
    )(agg, st, g, be, dinv, w1)


def _tc_f_body(agg_ref, st_ref, g_ref, be_ref, batch_ref,
               nn_ref, gr_ref, cnt_scr):
    i = pl.program_id(0)
    m = st_ref[0:1, :] * (1.0 / N)
    v = st_ref[1:2, :] * (1.0 / N) - m * m
    h = (agg_ref[...] - m) * lax.rsqrt(v + EPS) * g_ref[...] + be_ref[...]
    nrm = jnp.sqrt(jnp.sum(h * h, axis=1, keepdims=True))
    nn_ref[...] = h / jnp.maximum(nrm, 1e-12)
    oh = (batch_ref[...] == lax.broadcasted_iota(jnp.int32, (R, G), 1))
    ohf = oh.astype(jnp.float32)
    part = lax.dot_general(ohf, h, (((0,), (0,)), ((), ())),
                           preferred_element_type=jnp.float32)
    pc = lax.dot_general(ohf, jnp.ones((R, D), jnp.float32),
                         (((0,), (0,)), ((), ())),
                         preferred_element_type=jnp.float32)

    @pl.when(i == 0)
    def _():
        gr_ref[...] = jnp.zeros_like(gr_ref)
        cnt_scr[...] = jnp.zeros_like(cnt_scr)

    gr_ref[...] += part
    cnt_scr[...] += pc

    @pl.when(i == NBLK - 1)
    def _():
        gr_ref[...] = gr_ref[...] / jnp.maximum(cnt_scr[...], 1.0)


def _tc_final(agg, st, g, be, batch2):
    return pl.pallas_call(
        _tc_f_body,
        grid=(NBLK,),
        in_specs=[
            pl.BlockSpec((R, D), lambda i: (i, 0)),
            pl.BlockSpec((8, D), lambda i: (0, 0)),
            pl.BlockSpec((1, D), lambda i: (0, 0)),
            pl.BlockSpec((1, D), lambda i: (0, 0)),
            pl.BlockSpec((R, 1), lambda i: (i, 0)),
        ],
        out_specs=[
            pl.BlockSpec((R, D), lambda i: (i, 0)),
            pl.BlockSpec((G, D), lambda i: (0, 0)),
        ],
        out_shape=[
            jax.ShapeDtypeStruct((N, D), jnp.float32),
            jax.ShapeDtypeStruct((G, D), jnp.float32),
        ],
        scratch_shapes=[pltpu.VMEM((G, D), jnp.float32)],
    )(agg, st, g, be, batch2)


def kernel(x, edge_index, batch, emb_table, W0, b0, g0, be0, W1, b1, g1, be1):
    src = edge_index[0]
    dst = edge_index[1]
    pad = EPAD - E
    src_p = jnp.concatenate([src, jnp.zeros((pad,), src.dtype)])
    trash = N + (jnp.arange(pad, dtype=dst.dtype) % (NPAD - N))
    dst_p = jnp.concatenate([dst, trash])
    src3 = src_p.reshape(NW, NCH, CH)
    dst3 = dst_p.reshape(NW, NCH, CH)

    degp = _sc_deg(dst3)
    y0, dinv = _tc_a(x, emb_table, W0, degp)
    z0 = _sc_scatter(y0, src3, dst3)
    agg0, st0 = _tc_stats(z0, y0, dinv, b0.reshape(1, D))
    y1 = _tc_mid(agg0, st0, g0.reshape(1, D), be0.reshape(1, D), dinv, W1)
    z1 = _sc_scatter(y1, src3, dst3)
    agg1, st1 = _tc_stats(z1, y1, dinv, b1.reshape(1, D))
    node_norm, graph_rep = _tc_final(agg1, st1, g1.reshape(1, D),
                                     be1.reshape(1, D), batch.reshape(N, 1))
    return graph_rep, node_norm


# speedup vs baseline: 21.3657x; 2.2308x over previous
import functools

import jax
import jax.numpy as jnp
from jax import lax
from jax.experimental import pallas as pl
from jax.experimental.pallas import tpu as pltpu
from jax.experimental.pallas import tpu_sc as plsc

N = 10000
D = 128
E = 320000
G = 16
CARD = 128
EPS = 1e-5

NC, NS = 2, 16
NW = NC * NS
CH = 128
NCH = 80
EPW = CH * NCH
EPAD = EPW * NW
RPT = 632
NPAD = RPT * NS
HNCH = NCH // 2

R = 1000
NBLK = N // R

@functools.cache
def _mesh():
    return plsc.VectorSubcoreMesh(core_axis_name="c", subcore_axis_name="s",
                                  num_cores=NC, num_subcores=NS)


def _sc_deg_body(dst_hbm, ones_hbm, zero_hbm, out_hbm, dst_v, ones_v, deg_sh):
    c = lax.axis_index("c")
    s = lax.axis_index("s")
    wid = c * NS + s

    @pl.when(s == 0)
    def _():
        pltpu.sync_copy(zero_hbm, deg_sh)

    pltpu.sync_copy(ones_hbm, ones_v)
    pltpu.sync_copy(dst_hbm.at[wid], dst_v)
    plsc.subcore_barrier()

    def _chunk(j, _):
        pltpu.sync_copy(ones_v, deg_sh.at[dst_v.at[j]], add=True)
        return 0

    lax.fori_loop(0, NCH, _chunk, 0)
    plsc.subcore_barrier()

    @pl.when(s == 0)
    def _():
        pltpu.sync_copy(deg_sh, out_hbm.at[c])


@functools.cache
def _sc_deg_kernel():
    return pl.kernel(
        _sc_deg_body,
        out_type=jax.ShapeDtypeStruct((NC, NPAD, D), jnp.float32),
        mesh=_mesh(),
        scratch_types=[
            pltpu.VMEM((NCH, CH), jnp.int32),
            pltpu.VMEM((CH, D), jnp.float32),
            pltpu.VMEM_SHARED((NPAD, D), jnp.float32),
        ],
    )


def _sc_deg(dst3):
    ones_d = jnp.ones((CH, D), jnp.float32)
    zero_d = jnp.zeros((NPAD, D), jnp.float32)
    return _sc_deg_kernel()(dst3, ones_d, zero_d)


def _sc_scat_body(y_hbm, src_hbm, dst_hbm, zero_hbm, out_hbm,
                  rows_a, rows_b, src_v, dst_v, z_sh, sem_a, sem_b):
    c = lax.axis_index("c")
    s = lax.axis_index("s")
    wid = c * NS + s

    @pl.when(s == 0)
    def _():
        pltpu.sync_copy(zero_hbm, z_sh)

    plsc.subcore_barrier()

    for h in range(2):
        pltpu.sync_copy(src_hbm.at[wid, pl.ds(h * HNCH, HNCH)], src_v)
        pltpu.sync_copy(dst_hbm.at[wid, pl.ds(h * HNCH, HNCH)], dst_v)
        pltpu.async_copy(y_hbm.at[src_v.at[0]], rows_a, sem_a)

        def _step(j, _):
            pltpu.make_async_copy(y_hbm.at[src_v.at[2 * j]], rows_a,
                                  sem_a).wait()
            pltpu.async_copy(y_hbm.at[src_v.at[2 * j + 1]], rows_b, sem_b)
            pltpu.sync_copy(rows_a, z_sh.at[dst_v.at[2 * j]], add=True)
            pltpu.make_async_copy(y_hbm.at[src_v.at[2 * j + 1]], rows_b,
                                  sem_b).wait()

            @pl.when(j < HNCH // 2 - 1)
            def _():
                pltpu.async_copy(y_hbm.at[src_v.at[2 * j + 2]], rows_a, sem_a)

            pltpu.sync_copy(rows_b, z_sh.at[dst_v.at[2 * j + 1]], add=True)
            return 0

        lax.fori_loop(0, HNCH // 2, _step, 0)
    plsc.subcore_barrier()

    @pl.when(s == 0)
    def _():
        pltpu.sync_copy(z_sh, out_hbm.at[c])


@functools.cache
def _sc_scatter_kernel():
    return pl.kernel(
        _sc_scat_body,
        out_type=jax.ShapeDtypeStruct((NC, NPAD, D), jnp.float32),
        mesh=_mesh(),
        scratch_types=[
            pltpu.VMEM((CH, D), jnp.float32),
            pltpu.VMEM((CH, D), jnp.float32),
            pltpu.VMEM((HNCH, CH), jnp.int32),
            pltpu.VMEM((HNCH, CH), jnp.int32),
            pltpu.VMEM_SHARED((NPAD, D), jnp.float32),
            pltpu.SemaphoreType.DMA,
            pltpu.SemaphoreType.DMA,
        ],
    )


def _sc_scatter(y, src3, dst3):
    zero_d = jnp.zeros((NPAD, D), jnp.float32)
    return _sc_scatter_kernel()(y, src3, dst3, zero_d)


def _tc_a_body(x_ref, emb_ref, w0_ref, degp_ref, y0_ref, dinv_ref):
    xb = x_ref[...]
    oh = (xb == lax.broadcasted_iota(jnp.int32, (R, CARD), 1))
    h0 = jnp.dot(oh.astype(jnp.float32), emb_ref[...],
                 preferred_element_type=jnp.float32)
    xw = jnp.dot(h0, w0_ref[...], preferred_element_type=jnp.float32)
    deg = degp_ref[0, :, 0:1] + degp_ref[1, :, 0:1] + 1.0
    dinv = lax.rsqrt(deg)
    dinv_ref[...] = dinv
    y0_ref[...] = xw * dinv


def _tc_a(x, emb, w0, degp):
    return pl.pallas_call(
        _tc_a_body,
        grid=(NBLK,),
        in_specs=[
            pl.BlockSpec((R, 1), lambda i: (i, 0)),
            pl.BlockSpec((CARD, D), lambda i: (0, 0)),
            pl.BlockSpec((D, D), lambda i: (0, 0)),
            pl.BlockSpec((2, R, D), lambda i: (0, i, 0)),
        ],
        out_specs=[
            pl.BlockSpec((R, D), lambda i: (i, 0)),
            pl.BlockSpec((R, 1), lambda i: (i, 0)),
        ],
        out_shape=[
            jax.ShapeDtypeStruct((N, D), jnp.float32),
            jax.ShapeDtypeStruct((N, 1), jnp.float32),
        ],
    )(x, emb, w0, degp)


def _tc_s_body(zp_ref, y_ref, dinv_ref, b_ref, agg_ref, st_ref):
    agg = (zp_ref[0] + zp_ref[1] + y_ref[...]) * dinv_ref[...] + b_ref[...]
    agg_ref[...] = agg

    @pl.when(pl.program_id(0) == 0)
    def _():
        st_ref[...] = jnp.zeros_like(st_ref)

    st_ref[0:1, :] += jnp.sum(agg, axis=0, keepdims=True)
    st_ref[1:2, :] += jnp.sum(agg * agg, axis=0, keepdims=True)


def _tc_stats(zp, y, dinv, b):
    return pl.pallas_call(
        _tc_s_body,
        grid=(NBLK,),
        in_specs=[
            pl.BlockSpec((2, R, D), lambda i: (0, i, 0)),
            pl.BlockSpec((R, D), lambda i: (i, 0)),
            pl.BlockSpec((R, 1), lambda i: (i, 0)),
            pl.BlockSpec((1, D), lambda i: (0, 0)),
        ],
        out_specs=[
            pl.BlockSpec((R, D), lambda i: (i, 0)),
            pl.BlockSpec((8, D), lambda i: (0, 0)),
        ],
        out_shape=[
            jax.ShapeDtypeStruct((N, D), jnp.float32),
            jax.ShapeDtypeStruct((8, D), jnp.float32),
        ],
    )(zp, y, dinv, b)


def _tc_m_body(agg_ref, st_ref, g_ref, be_ref, dinv_ref, w_ref, y1_ref):
    m = st_ref[0:1, :] * (1.0 / N)
    v = st_ref[1:2, :] * (1.0 / N) - m * m
    rstd = lax.rsqrt(v + EPS)
    h = jnp.maximum((agg_ref[...] - m) * rstd * g_ref[...] + be_ref[...], 0.0)
    y1_ref[...] = jnp.dot(h, w_ref[...],
                          preferred_element_type=jnp.float32) * dinv_ref[...]


def _tc_mid(agg, st, g, be, dinv, w1):
    return pl.pallas_call(
        _tc_m_body,
        grid=(NBLK,),
        in_specs=[
            pl.BlockSpec((R, D), lambda i: (i, 0)),
            pl.BlockSpec((8, D), lambda i: (0, 0)),
            pl.BlockSpec((1, D), lambda i: (0, 0)),
            pl.BlockSpec((1, D), lambda i: (0, 0)),
            pl.BlockSpec((R, 1), lambda i: (i, 0)),
            pl.BlockSpec((D, D), lambda i: (0, 0)),
        ],
        out_specs=pl.BlockSpec((R, D), lambda i: (i, 0)),
        out_shape=jax.ShapeDtypeStruct((N, D), jnp.float32),
    )(agg, st, g, be, dinv, w1)


def _tc_f_body(agg_ref, st_ref, g_ref, be_ref, batch_ref,
               nn_ref, gr_ref, cnt_scr):
    i = pl.program_id(0)
    m = st_ref[0:1, :] * (1.0 / N)
    v = st_ref[1:2, :] * (1.0 / N) - m * m
    h = (agg_ref[...] - m) * lax.rsqrt(v + EPS) * g_ref[...] + be_ref[...]
    nrm = jnp.sqrt(jnp.sum(h * h, axis=1, keepdims=True))
    nn_ref[...] = h / jnp.maximum(nrm, 1e-12)
    oh = (batch_ref[...] == lax.broadcasted_iota(jnp.int32, (R, G), 1))
    ohf = oh.astype(jnp.float32)
    part = lax.dot_general(ohf, h, (((0,), (0,)), ((), ())),
                           preferred_element_type=jnp.float32)
    pc = lax.dot_general(ohf, jnp.ones((R, D), jnp.float32),
                         (((0,), (0,)), ((), ())),
                         preferred_element_type=jnp.float32)

    @pl.when(i == 0)
    def _():
        gr_ref[...] = jnp.zeros_like(gr_ref)
        cnt_scr[...] = jnp.zeros_like(cnt_scr)

    gr_ref[...] += part
    cnt_scr[...] += pc

    @pl.when(i == NBLK - 1)
    def _():
        gr_ref[...] = gr_ref[...] / jnp.maximum(cnt_scr[...], 1.0)


def _tc_final(agg, st, g, be, batch2):
    return pl.pallas_call(
        _tc_f_body,
        grid=(NBLK,),
        in_specs=[
            pl.BlockSpec((R, D), lambda i: (i, 0)),
            pl.BlockSpec((8, D), lambda i: (0, 0)),
            pl.BlockSpec((1, D), lambda i: (0, 0)),
            pl.BlockSpec((1, D), lambda i: (0, 0)),
            pl.BlockSpec((R, 1), lambda i: (i, 0)),
        ],
        out_specs=[
            pl.BlockSpec((R, D), lambda i: (i, 0)),
            pl.BlockSpec((G, D), lambda i: (0, 0)),
        ],
        out_shape=[
            jax.ShapeDtypeStruct((N, D), jnp.float32),
            jax.ShapeDtypeStruct((G, D), jnp.float32),
        ],
        scratch_shapes=[pltpu.VMEM((G, D), jnp.float32)],
    )(agg, st, g, be, batch2)


def kernel(x, edge_index, batch, emb_table, W0, b0, g0, be0, W1, b1, g1, be1):
    src = edge_index[0]
    dst = edge_index[1]
    ppw = EPW - E // NW
    pidx = jnp.arange(NW * ppw, dtype=dst.dtype).reshape(NW, ppw)
    pad_src = pidx % N
    pad_dst = N + pidx % (NPAD - N)
    src3 = jnp.concatenate([src.reshape(NW, E // NW), pad_src],
                           axis=1).reshape(NW, NCH, CH)
    dst3 = jnp.concatenate([dst.reshape(NW, E // NW), pad_dst],
                           axis=1).reshape(NW, NCH, CH)

    degp = _sc_deg(dst3)
    y0, dinv = _tc_a(x, emb_table, W0, degp)
    z0 = _sc_scatter(y0, src3, dst3)
    agg0, st0 = _tc_stats(z0, y0, dinv, b0.reshape(1, D))
    y1 = _tc_mid(agg0, st0, g0.reshape(1, D), be0.reshape(1, D), dinv, W1)
    z1 = _sc_scatter(y1, src3, dst3)
    agg1, st1 = _tc_stats(z1, y1, dinv, b1.reshape(1, D))
    node_norm, graph_rep = _tc_final(agg1, st1, g1.reshape(1, D),
                                     be1.reshape(1, D), batch.reshape(N, 1))
    return graph_rep, node_norm
